# Initial kernel scaffold; baseline (speedup 1.0000x reference)
import jax
import jax.numpy as jnp
from jax.experimental import pallas as pl


def kernel(x, edge_index, edge_attr, nn1_w, nn1_b, root1, bias1, nn2_w, nn2_b, root2, bias2, nn3_w, nn3_b, root3, bias3, nn4_w, nn4_b, root4, bias4):
    raise NotImplementedError("write your pallas kernel here")


# speedup vs baseline: 1.8072x; 1.8072x over previous
import functools

import jax
import jax.numpy as jnp
from jax import lax
from jax.experimental import pallas as pl
from jax.experimental.pallas import tpu as pltpu
from jax.experimental.pallas import tpu_sc as plsc

NN = 10000
EE = 80000
CIN = 32
CEF = 16
CH = 16

NW = 32
PW = 2560
NCHUNK = 20
E_PAD = NW * PW
N_PAD = 10240
PS = N_PAD // 16

def _mesh_sc():
    return plsc.VectorSubcoreMesh(core_axis_name="c", subcore_axis_name="s",
                                  num_cores=2, num_subcores=16)


def _sc_gather(h, src, ch):
    @functools.partial(
        pl.kernel,
        out_type=jax.ShapeDtypeStruct((E_PAD, ch), jnp.float32),
        mesh=_mesh_sc(),
        compiler_params=pltpu.CompilerParams(use_tc_tiling_on_sc=False),
        scratch_types=[
            pltpu.VMEM((PW,), jnp.int32),
            pltpu.VMEM((PW, ch), jnp.float32),
            pltpu.SemaphoreType.DMA,
        ],
    )
    def k(h_hbm, src_hbm, out_hbm, idx_v, rows_v, sem):
        wid = lax.axis_index("s") * 2 + lax.axis_index("c")
        base = wid * PW
        pltpu.sync_copy(src_hbm.at[pl.ds(base, PW)], idx_v)
        copies = []
        for j in range(NCHUNK):
            copies.append(pltpu.async_copy(
                h_hbm.at[idx_v.at[pl.ds(j * 128, 128)]],
                rows_v.at[pl.ds(j * 128, 128)], sem))
        for cp in copies:
            cp.wait()
        pltpu.sync_copy(rows_v, out_hbm.at[pl.ds(base, PW)])

    return k(h, src)


def _sc_scatter_cnt(msg, dst3, zeros, ones):
    @functools.partial(
        pl.kernel,
        out_type=[jax.ShapeDtypeStruct((2, N_PAD, CH), jnp.float32),
                  jax.ShapeDtypeStruct((2, N_PAD, CH), jnp.float32)],
        mesh=_mesh_sc(),
        compiler_params=pltpu.CompilerParams(use_tc_tiling_on_sc=False),
        scratch_types=[
            pltpu.VMEM((NCHUNK, 128), jnp.int32),
            pltpu.VMEM((PW, CH), jnp.float32),
            pltpu.VMEM((128, CH), jnp.float32),
            pltpu.VMEM_SHARED((N_PAD, CH), jnp.float32),
            pltpu.VMEM_SHARED((N_PAD, CH), jnp.float32),
        ],
    )
    def k(msg_hbm, dst_hbm, z_hbm, o_hbm, out_hbm, cnt_hbm,
          idx_v, rows_v, ones_v, acc_sh, cacc_sh):
        c = lax.axis_index("c")
        s = lax.axis_index("s")
        wid = s * 2 + c
        pltpu.sync_copy(z_hbm, acc_sh.at[pl.ds(s * PS, PS)])
        pltpu.sync_copy(z_hbm, cacc_sh.at[pl.ds(s * PS, PS)])
        pltpu.sync_copy(dst_hbm.at[wid], idx_v)
        pltpu.sync_copy(msg_hbm.at[pl.ds(wid * PW, PW)], rows_v)
        pltpu.sync_copy(o_hbm, ones_v)
        plsc.subcore_barrier()
        for j in range(NCHUNK):
            pltpu.sync_copy(rows_v.at[pl.ds(j * 128, 128)],
                            acc_sh.at[idx_v.at[j]], add=True)
            pltpu.sync_copy(ones_v, cacc_sh.at[idx_v.at[j]], add=True)
        plsc.subcore_barrier()
        pltpu.sync_copy(acc_sh.at[pl.ds(s * PS, PS)],
                        out_hbm.at[c, pl.ds(s * PS, PS)])
        pltpu.sync_copy(cacc_sh.at[pl.ds(s * PS, PS)],
                        cnt_hbm.at[c, pl.ds(s * PS, PS)])

    return k(msg, dst3, zeros, ones)


def _sc_scatter(msg, dst3, zeros):
    @functools.partial(
        pl.kernel,
        out_type=jax.ShapeDtypeStruct((2, N_PAD, CH), jnp.float32),
        mesh=_mesh_sc(),
        compiler_params=pltpu.CompilerParams(use_tc_tiling_on_sc=False),
        scratch_types=[
            pltpu.VMEM((NCHUNK, 128), jnp.int32),
            pltpu.VMEM((PW, CH), jnp.float32),
            pltpu.VMEM_SHARED((N_PAD, CH), jnp.float32),
        ],
    )
    def k(msg_hbm, dst_hbm, z_hbm, out_hbm, idx_v, rows_v, acc_sh):
        c = lax.axis_index("c")
        s = lax.axis_index("s")
        wid = s * 2 + c
        pltpu.sync_copy(z_hbm, acc_sh.at[pl.ds(s * PS, PS)])
        pltpu.sync_copy(dst_hbm.at[wid], idx_v)
        pltpu.sync_copy(msg_hbm.at[pl.ds(wid * PW, PW)], rows_v)
        plsc.subcore_barrier()
        for j in range(NCHUNK):
            pltpu.sync_copy(rows_v.at[pl.ds(j * 128, 128)],
                            acc_sh.at[idx_v.at[j]], add=True)
        plsc.subcore_barrier()
        pltpu.sync_copy(acc_sh.at[pl.ds(s * PS, PS)],
                        out_hbm.at[c, pl.ds(s * PS, PS)])

    return k(msg, dst3, zeros)


def _tc_msg(hs, ea, w3, nbm, ch, tile_e=2048):
    grid = E_PAD // tile_e

    def body(hs_ref, ea_ref, w_ref, nb_ref, out_ref):
        hsv = hs_ref[...]
        eav = ea_ref[...]
        acc = jnp.dot(hsv, nb_ref[...], preferred_element_type=jnp.float32)
        for f in range(CEF):
            acc = acc + eav[:, f:f + 1] * jnp.dot(
                hsv, w_ref[f], preferred_element_type=jnp.float32)
        out_ref[...] = acc

    return pl.pallas_call(
        body,
        grid=(grid,),
        in_specs=[
            pl.BlockSpec((tile_e, ch), lambda i: (i, 0)),
            pl.BlockSpec((tile_e, CEF), lambda i: (i, 0)),
            pl.BlockSpec((CEF, ch, CH), lambda i: (0, 0, 0)),
            pl.BlockSpec((ch, CH), lambda i: (0, 0)),
        ],
        out_specs=pl.BlockSpec((tile_e, CH), lambda i: (i, 0)),
        out_shape=jax.ShapeDtypeStruct((E_PAD, CH), jnp.float32),
    )(hs, ea, w3, nbm)


def _tc_update(p, cntp, h, root, bias, relu, ch, tile_n=1024):
    grid = N_PAD // tile_n

    def body(p_ref, c_ref, h_ref, r_ref, b_ref, out_ref):
        summed = p_ref[0] + p_ref[1]
        cnt = c_ref[0][:, :1] + c_ref[1][:, :1]
        inv = 1.0 / jnp.maximum(cnt, 1.0)
        z = summed * inv + jnp.dot(
            h_ref[...], r_ref[...], preferred_element_type=jnp.float32) + b_ref[...]
        out_ref[...] = jnp.maximum(z, 0.0) if relu else z

    return pl.pallas_call(
        body,
        grid=(grid,),
        in_specs=[
            pl.BlockSpec((2, tile_n, CH), lambda i: (0, i, 0)),
            pl.BlockSpec((2, tile_n, CH), lambda i: (0, i, 0)),
            pl.BlockSpec((tile_n, ch), lambda i: (i, 0)),
            pl.BlockSpec((ch, CH), lambda i: (0, 0)),
            pl.BlockSpec((1, CH), lambda i: (0, 0)),
        ],
        out_specs=pl.BlockSpec((tile_n, CH), lambda i: (i, 0)),
        out_shape=jax.ShapeDtypeStruct((N_PAD, CH), jnp.float32),
    )(p, cntp, h, root, bias)


def _prep_w(nw, nb, ch, out_ch):
    w3 = nw.T.reshape(CEF, ch, out_ch)
    nbm = nb.reshape(ch, out_ch)
    if out_ch < CH:
        w3 = jnp.pad(w3, ((0, 0), (0, 0), (0, CH - out_ch)))
        nbm = jnp.pad(nbm, ((0, 0), (0, CH - out_ch)))
    return w3, nbm


def kernel(x, edge_index, edge_attr,
           nn1_w, nn1_b, root1, bias1,
           nn2_w, nn2_b, root2, bias2,
           nn3_w, nn3_b, root3, bias3,
           nn4_w, nn4_b, root4, bias4):
    src = jnp.pad(edge_index[0], (0, E_PAD - EE))
    dst3 = jnp.pad(edge_index[1], (0, E_PAD - EE),
                   constant_values=NN).reshape(NW, NCHUNK, 128)
    ea = jnp.pad(edge_attr, ((0, E_PAD - EE), (0, 0)))
    zeros = jnp.zeros((PS, CH), jnp.float32)
    ones = jnp.ones((128, CH), jnp.float32)

    w1, nbm1 = _prep_w(nn1_w, nn1_b, CIN, CH)
    w2, nbm2 = _prep_w(nn2_w, nn2_b, CH, CH)
    w3_, nbm3 = _prep_w(nn3_w, nn3_b, CH, CH)
    w4, nbm4 = _prep_w(nn4_w, nn4_b, CH, 8)
    r4 = jnp.pad(root4, ((0, 0), (0, CH - 8)))
    b4 = jnp.pad(bias4, (0, CH - 8))

    h = jnp.pad(x, ((0, N_PAD - NN), (0, 0)))
    cntp = None
    layers = [
        (w1, nbm1, root1, bias1, CIN, True),
        (w2, nbm2, root2, bias2, CH, True),
        (w3_, nbm3, root3, bias3, CH, True),
        (w4, nbm4, r4, b4, CH, False),
    ]
    for li, (w3v, nbm, root, bias, ch, relu) in enumerate(layers):
        hs = _sc_gather(h, src, ch)
        msg = _tc_msg(hs, ea, w3v, nbm, ch)
        if li == 0:
            p, cntp = _sc_scatter_cnt(msg, dst3, zeros, ones)
        else:
            p = _sc_scatter(msg, dst3, zeros)
        h = _tc_update(p, cntp, h, root.astype(jnp.float32),
                       bias.reshape(1, CH).astype(jnp.float32), relu, ch)

    return h[:NN, :8]


# speedup vs baseline: 2.8300x; 1.5659x over previous
import functools

import jax
import jax.numpy as jnp
from jax import lax
from jax.experimental import pallas as pl
from jax.experimental.pallas import tpu as pltpu
from jax.experimental.pallas import tpu_sc as plsc

NN = 10000
EE = 80000
CIN = 32
CEF = 16
CH = 16

NW = 32
PW = 2560
NCHUNK = 20
E_PAD = NW * PW
N_PAD = 10240
PS = N_PAD // 16

def _mesh_sc():
    return plsc.VectorSubcoreMesh(core_axis_name="c", subcore_axis_name="s",
                                  num_cores=2, num_subcores=16)


def _sc_gather(h, src, ch):
    @functools.partial(
        pl.kernel,
        out_type=jax.ShapeDtypeStruct((E_PAD, ch), jnp.float32),
        mesh=_mesh_sc(),
        compiler_params=pltpu.CompilerParams(use_tc_tiling_on_sc=False),
        scratch_types=[
            pltpu.VMEM((PW,), jnp.int32),
            pltpu.VMEM((PW, ch), jnp.float32),
            pltpu.SemaphoreType.DMA,
        ],
    )
    def k(h_hbm, src_hbm, out_hbm, idx_v, rows_v, sem):
        wid = lax.axis_index("s") * 2 + lax.axis_index("c")
        base = wid * PW
        pltpu.sync_copy(src_hbm.at[pl.ds(base, PW)], idx_v)
        copies = []
        for j in range(NCHUNK):
            copies.append(pltpu.async_copy(
                h_hbm.at[idx_v.at[pl.ds(j * 128, 128)]],
                rows_v.at[pl.ds(j * 128, 128)], sem))
        for cp in copies:
            cp.wait()
        pltpu.sync_copy(rows_v, out_hbm.at[pl.ds(base, PW)])

    return k(h, src)


def _sc_scatter_cnt(msg, dst3, zeros, ones):
    @functools.partial(
        pl.kernel,
        out_type=[jax.ShapeDtypeStruct((2, N_PAD, CH), jnp.float32),
                  jax.ShapeDtypeStruct((2, N_PAD, CH), jnp.float32)],
        mesh=_mesh_sc(),
        compiler_params=pltpu.CompilerParams(use_tc_tiling_on_sc=False),
        scratch_types=[
            pltpu.VMEM((NCHUNK, 128), jnp.int32),
            pltpu.VMEM((PW, CH), jnp.float32),
            pltpu.VMEM((128, CH), jnp.float32),
            pltpu.VMEM_SHARED((N_PAD, CH), jnp.float32),
            pltpu.VMEM_SHARED((N_PAD, CH), jnp.float32),
        ],
    )
    def k(msg_hbm, dst_hbm, z_hbm, o_hbm, out_hbm, cnt_hbm,
          idx_v, rows_v, ones_v, acc_sh, cacc_sh):
        c = lax.axis_index("c")
        s = lax.axis_index("s")
        wid = s * 2 + c
        pltpu.sync_copy(z_hbm, acc_sh.at[pl.ds(s * PS, PS)])
        pltpu.sync_copy(z_hbm, cacc_sh.at[pl.ds(s * PS, PS)])
        pltpu.sync_copy(dst_hbm.at[wid], idx_v)
        pltpu.sync_copy(msg_hbm.at[pl.ds(wid * PW, PW)], rows_v)
        pltpu.sync_copy(o_hbm, ones_v)
        plsc.subcore_barrier()
        for j in range(NCHUNK):
            pltpu.sync_copy(rows_v.at[pl.ds(j * 128, 128)],
                            acc_sh.at[idx_v.at[j]], add=True)
            pltpu.sync_copy(ones_v, cacc_sh.at[idx_v.at[j]], add=True)
        plsc.subcore_barrier()
        pltpu.sync_copy(acc_sh.at[pl.ds(s * PS, PS)],
                        out_hbm.at[c, pl.ds(s * PS, PS)])
        pltpu.sync_copy(cacc_sh.at[pl.ds(s * PS, PS)],
                        cnt_hbm.at[c, pl.ds(s * PS, PS)])

    return k(msg, dst3, zeros, ones)


def _sc_scatter(msg, dst3, zeros):
    @functools.partial(
        pl.kernel,
        out_type=jax.ShapeDtypeStruct((2, N_PAD, CH), jnp.float32),
        mesh=_mesh_sc(),
        compiler_params=pltpu.CompilerParams(use_tc_tiling_on_sc=False),
        scratch_types=[
            pltpu.VMEM((NCHUNK, 128), jnp.int32),
            pltpu.VMEM((PW, CH), jnp.float32),
            pltpu.VMEM_SHARED((N_PAD, CH), jnp.float32),
        ],
    )
    def k(msg_hbm, dst_hbm, z_hbm, out_hbm, idx_v, rows_v, acc_sh):
        c = lax.axis_index("c")
        s = lax.axis_index("s")
        wid = s * 2 + c
        pltpu.sync_copy(z_hbm, acc_sh.at[pl.ds(s * PS, PS)])
        pltpu.sync_copy(dst_hbm.at[wid], idx_v)
        pltpu.sync_copy(msg_hbm.at[pl.ds(wid * PW, PW)], rows_v)
        plsc.subcore_barrier()
        for j in range(NCHUNK):
            pltpu.sync_copy(rows_v.at[pl.ds(j * 128, 128)],
                            acc_sh.at[idx_v.at[j]], add=True)
        plsc.subcore_barrier()
        pltpu.sync_copy(acc_sh.at[pl.ds(s * PS, PS)],
                        out_hbm.at[c, pl.ds(s * PS, PS)])

    return k(msg, dst3, zeros)


def _tc_msg(hs, ea, wstk, rmat, ch, tile_e=2048):
    grid = E_PAD // tile_e

    def body(hs_ref, ea_ref, w_ref, r_ref, out_ref):
        t = jnp.dot(hs_ref[...], w_ref[...], preferred_element_type=jnp.float32)
        eae = jnp.dot(ea_ref[...], r_ref[...], preferred_element_type=jnp.float32)
        p = t[:, :256] * eae
        p = p[:, :128] + p[:, 128:256]
        p = p[:, :64] + p[:, 64:128]
        p = p[:, :32] + p[:, 32:64]
        out_ref[...] = p[:, :16] + p[:, 16:32] + t[:, 256:272]

    return pl.pallas_call(
        body,
        grid=(grid,),
        in_specs=[
            pl.BlockSpec((tile_e, ch), lambda i: (i, 0)),
            pl.BlockSpec((tile_e, CEF), lambda i: (i, 0)),
            pl.BlockSpec((ch, 272), lambda i: (0, 0)),
            pl.BlockSpec((CEF, 256), lambda i: (0, 0)),
        ],
        out_specs=pl.BlockSpec((tile_e, CH), lambda i: (i, 0)),
        out_shape=jax.ShapeDtypeStruct((E_PAD, CH), jnp.float32),
    )(hs, ea, wstk, rmat)


def _tc_update(p, cntp, h, root, bias, relu, ch, tile_n=1024):
    grid = N_PAD // tile_n

    def body(p_ref, c_ref, h_ref, r_ref, b_ref, out_ref):
        summed = p_ref[0] + p_ref[1]
        cnt = c_ref[0][:, :1] + c_ref[1][:, :1]
        inv = 1.0 / jnp.maximum(cnt, 1.0)
        z = summed * inv + jnp.dot(
            h_ref[...], r_ref[...], preferred_element_type=jnp.float32) + b_ref[...]
        out_ref[...] = jnp.maximum(z, 0.0) if relu else z

    return pl.pallas_call(
        body,
        grid=(grid,),
        in_specs=[
            pl.BlockSpec((2, tile_n, CH), lambda i: (0, i, 0)),
            pl.BlockSpec((2, tile_n, CH), lambda i: (0, i, 0)),
            pl.BlockSpec((tile_n, ch), lambda i: (i, 0)),
            pl.BlockSpec((ch, CH), lambda i: (0, 0)),
            pl.BlockSpec((1, CH), lambda i: (0, 0)),
        ],
        out_specs=pl.BlockSpec((tile_n, CH), lambda i: (i, 0)),
        out_shape=jax.ShapeDtypeStruct((N_PAD, CH), jnp.float32),
    )(p, cntp, h, root, bias)


def _prep_w(nw, nb, ch, out_ch):
    w3 = nw.T.reshape(CEF, ch, out_ch)
    nbm = nb.reshape(ch, out_ch)
    if out_ch < CH:
        w3 = jnp.pad(w3, ((0, 0), (0, 0), (0, CH - out_ch)))
        nbm = jnp.pad(nbm, ((0, 0), (0, CH - out_ch)))
    wstk = jnp.concatenate(
        [w3.transpose(1, 0, 2).reshape(ch, CEF * CH), nbm], axis=1)
    return wstk


def kernel(x, edge_index, edge_attr,
           nn1_w, nn1_b, root1, bias1,
           nn2_w, nn2_b, root2, bias2,
           nn3_w, nn3_b, root3, bias3,
           nn4_w, nn4_b, root4, bias4):
    src = jnp.pad(edge_index[0], (0, E_PAD - EE))
    dst3 = jnp.pad(edge_index[1], (0, E_PAD - EE),
                   constant_values=NN).reshape(NW, NCHUNK, 128)
    ea = jnp.pad(edge_attr, ((0, E_PAD - EE), (0, 0)))
    zeros = jnp.zeros((PS, CH), jnp.float32)
    ones = jnp.ones((128, CH), jnp.float32)

    w1 = _prep_w(nn1_w, nn1_b, CIN, CH)
    w2 = _prep_w(nn2_w, nn2_b, CH, CH)
    w3_ = _prep_w(nn3_w, nn3_b, CH, CH)
    w4 = _prep_w(nn4_w, nn4_b, CH, 8)
    r4 = jnp.pad(root4, ((0, 0), (0, CH - 8)))
    b4 = jnp.pad(bias4, (0, CH - 8))
    rmat = jnp.repeat(jnp.eye(CEF, dtype=jnp.float32), CH, axis=1)

    h = jnp.pad(x, ((0, N_PAD - NN), (0, 0)))
    cntp = None
    layers = [
        (w1, root1, bias1, CIN, True),
        (w2, root2, bias2, CH, True),
        (w3_, root3, bias3, CH, True),
        (w4, r4, b4, CH, False),
    ]
    for li, (wstk, root, bias, ch, relu) in enumerate(layers):
        hs = _sc_gather(h, src, ch)
        msg = _tc_msg(hs, ea, wstk, rmat, ch)
        if li == 0:
            p, cntp = _sc_scatter_cnt(msg, dst3, zeros, ones)
        else:
            p = _sc_scatter(msg, dst3, zeros)
        h = _tc_update(p, cntp, h, root.astype(jnp.float32),
                       bias.reshape(1, CH).astype(jnp.float32), relu, ch)

    return h[:NN, :8]


# speedup vs baseline: 2.9557x; 1.0444x over previous
import functools

import jax
import jax.numpy as jnp
from jax import lax
from jax.experimental import pallas as pl
from jax.experimental.pallas import tpu as pltpu
from jax.experimental.pallas import tpu_sc as plsc

NN = 10000
EE = 80000
CIN = 32
CEF = 16
CH = 16

NW = 32
PW = 2560
E_PAD = NW * PW
PS16 = E_PAD // 16
SCH = PS16 // 128
N_PAD = 10240
PS = N_PAD // 16
UPS = N_PAD // 32


def _mesh_sc():
    return plsc.VectorSubcoreMesh(core_axis_name="c", subcore_axis_name="s",
                                  num_cores=2, num_subcores=16)


_SC_PARAMS = dict(
    compiler_params=pltpu.CompilerParams(use_tc_tiling_on_sc=False))


def _sc_gather(h, src, ch):
    @functools.partial(
        pl.kernel,
        out_type=jax.ShapeDtypeStruct((E_PAD, ch), jnp.float32),
        mesh=_mesh_sc(),
        scratch_types=[
            pltpu.VMEM((PW,), jnp.int32),
            pltpu.VMEM((PW, ch), jnp.float32),
            pltpu.SemaphoreType.DMA,
        ],
        **_SC_PARAMS,
    )
    def k(h_hbm, src_hbm, out_hbm, idx_v, rows_v, sem):
        wid = lax.axis_index("s") * 2 + lax.axis_index("c")
        base = wid * PW
        pltpu.sync_copy(src_hbm.at[pl.ds(base, PW)], idx_v)
        copies = []
        for j in range(PW // 128):
            copies.append(pltpu.async_copy(
                h_hbm.at[idx_v.at[pl.ds(j * 128, 128)]],
                rows_v.at[pl.ds(j * 128, 128)], sem))
        for cp in copies:
            cp.wait()
        pltpu.sync_copy(rows_v, out_hbm.at[pl.ds(base, PW)])

    return k(h, src)


def _sc_cnt_inv(dst4, ones, zeros):
    @functools.partial(
        pl.kernel,
        out_type=jax.ShapeDtypeStruct((N_PAD, CH), jnp.float32),
        mesh=_mesh_sc(),
        scratch_types=[
            pltpu.VMEM((SCH, 128), jnp.int32),
            pltpu.VMEM((128, CH), jnp.float32),
            pltpu.VMEM((UPS, CH), jnp.float32),
            pltpu.VMEM_SHARED((N_PAD, CH), jnp.float32),
        ],
        **_SC_PARAMS,
    )
    def k(dst_hbm, o_hbm, z_hbm, ci_hbm, idx_v, ones_v, buf_v, acc_sh):
        c = lax.axis_index("c")
        s = lax.axis_index("s")
        pltpu.sync_copy(z_hbm, acc_sh.at[pl.ds(s * PS, PS)])
        pltpu.sync_copy(dst_hbm.at[s], idx_v)
        pltpu.sync_copy(o_hbm, ones_v)
        plsc.subcore_barrier()

        def chunk5(step, _):
            for u in range(5):
                j = step * 5 + u
                pltpu.sync_copy(ones_v, acc_sh.at[idx_v.at[j]], add=True)
            return 0

        lax.fori_loop(0, SCH // 5, chunk5, 0)
        plsc.subcore_barrier()
        nbase = c * (N_PAD // 2) + s * UPS
        pltpu.sync_copy(acc_sh.at[pl.ds(nbase, UPS)], buf_v)

        def row(i, _):
            buf_v[i] = 1.0 / jnp.maximum(buf_v[i], 1.0)
            return 0

        lax.fori_loop(0, UPS, row, 0)
        pltpu.sync_copy(buf_v, ci_hbm.at[pl.ds(nbase, UPS)])

    return k(dst4, ones, zeros)


def _sc_scatter_update(msg, dst4, zeros, ci, hroot, relu):
    @functools.partial(
        pl.kernel,
        out_type=jax.ShapeDtypeStruct((N_PAD, CH), jnp.float32),
        mesh=_mesh_sc(),
        scratch_types=[
            pltpu.VMEM((SCH, 128), jnp.int32),
            pltpu.VMEM((PS16, CH), jnp.float32),
            pltpu.VMEM((UPS, CH), jnp.float32),
            pltpu.VMEM((UPS, CH), jnp.float32),
            pltpu.VMEM_SHARED((N_PAD, CH), jnp.float32),
        ],
        **_SC_PARAMS,
    )
    def k(msg_hbm, dst_hbm, z_hbm, ci_hbm, hr_hbm, out_hbm,
          idx_v, rows_v, a_v, b_v, acc_sh):
        c = lax.axis_index("c")
        s = lax.axis_index("s")
        pltpu.sync_copy(z_hbm, acc_sh.at[pl.ds(s * PS, PS)])
        pltpu.sync_copy(dst_hbm.at[s], idx_v)
        pltpu.sync_copy(msg_hbm.at[pl.ds(s * PS16, PS16)], rows_v)
        plsc.subcore_barrier()

        def chunk5(step, _):
            for u in range(5):
                j = step * 5 + u
                pltpu.sync_copy(rows_v.at[pl.ds(j * 128, 128)],
                                acc_sh.at[idx_v.at[j]], add=True)
            return 0

        lax.fori_loop(0, SCH // 5, chunk5, 0)
        plsc.subcore_barrier()
        nbase = c * (N_PAD // 2) + s * UPS
        pltpu.sync_copy(acc_sh.at[pl.ds(nbase, UPS)], a_v)
        pltpu.sync_copy(ci_hbm.at[pl.ds(nbase, UPS)], b_v)

        def row_mul(i, _):
            a_v[i] = a_v[i] * b_v[i]
            return 0

        lax.fori_loop(0, UPS, row_mul, 0)
        pltpu.sync_copy(hr_hbm.at[pl.ds(nbase, UPS)], b_v)

        if relu:
            def row_add(i, _):
                a_v[i] = jnp.maximum(a_v[i] + b_v[i], 0.0)
                return 0
        else:
            def row_add(i, _):
                a_v[i] = a_v[i] + b_v[i]
                return 0

        lax.fori_loop(0, UPS, row_add, 0)
        pltpu.sync_copy(a_v, out_hbm.at[pl.ds(nbase, UPS)])

    return k(msg, dst4, zeros, ci, hroot)


def _tc_msg(hs, ea, wstk, rmat, h, root, bias, ch, tile_e=2048):
    grid = E_PAD // tile_e
    tile_n = N_PAD // grid

    def body(hs_ref, ea_ref, w_ref, r_ref, h_ref, rt_ref, b_ref,
             out_ref, hr_ref):
        t = jnp.dot(hs_ref[...], w_ref[...], preferred_element_type=jnp.float32)
        eae = jnp.dot(ea_ref[...], r_ref[...], preferred_element_type=jnp.float32)
        p = t[:, :256] * eae
        p = p[:, :128] + p[:, 128:256]
        p = p[:, :64] + p[:, 64:128]
        p = p[:, :32] + p[:, 32:64]
        out_ref[...] = p[:, :16] + p[:, 16:32] + t[:, 256:272]
        hr_ref[...] = jnp.dot(
            h_ref[...], rt_ref[...], preferred_element_type=jnp.float32) + b_ref[...]

    return pl.pallas_call(
        body,
        grid=(grid,),
        in_specs=[
            pl.BlockSpec((tile_e, ch), lambda i: (i, 0)),
            pl.BlockSpec((tile_e, CEF), lambda i: (i, 0)),
            pl.BlockSpec((ch, 272), lambda i: (0, 0)),
            pl.BlockSpec((CEF, 256), lambda i: (0, 0)),
            pl.BlockSpec((tile_n, ch), lambda i: (i, 0)),
            pl.BlockSpec((ch, CH), lambda i: (0, 0)),
            pl.BlockSpec((1, CH), lambda i: (0, 0)),
        ],
        out_specs=[
            pl.BlockSpec((tile_e, CH), lambda i: (i, 0)),
            pl.BlockSpec((tile_n, CH), lambda i: (i, 0)),
        ],
        out_shape=[
            jax.ShapeDtypeStruct((E_PAD, CH), jnp.float32),
            jax.ShapeDtypeStruct((N_PAD, CH), jnp.float32),
        ],
    )(hs, ea, wstk, rmat, h, root, bias)


def _prep_w(nw, nb, ch, out_ch):
    w3 = nw.T.reshape(CEF, ch, out_ch)
    nbm = nb.reshape(ch, out_ch)
    if out_ch < CH:
        w3 = jnp.pad(w3, ((0, 0), (0, 0), (0, CH - out_ch)))
        nbm = jnp.pad(nbm, ((0, 0), (0, CH - out_ch)))
    wstk = jnp.concatenate(
        [w3.transpose(1, 0, 2).reshape(ch, CEF * CH), nbm], axis=1)
    return wstk


def kernel(x, edge_index, edge_attr,
           nn1_w, nn1_b, root1, bias1,
           nn2_w, nn2_b, root2, bias2,
           nn3_w, nn3_b, root3, bias3,
           nn4_w, nn4_b, root4, bias4):
    src = jnp.pad(edge_index[0], (0, E_PAD - EE))
    dst4 = jnp.pad(edge_index[1], (0, E_PAD - EE),
                   constant_values=NN).reshape(16, SCH, 128)
    ea = jnp.pad(edge_attr, ((0, E_PAD - EE), (0, 0)))
    zeros = jnp.zeros((PS, CH), jnp.float32)
    ones = jnp.ones((128, CH), jnp.float32)

    w1 = _prep_w(nn1_w, nn1_b, CIN, CH)
    w2 = _prep_w(nn2_w, nn2_b, CH, CH)
    w3_ = _prep_w(nn3_w, nn3_b, CH, CH)
    w4 = _prep_w(nn4_w, nn4_b, CH, 8)
    r4 = jnp.pad(root4, ((0, 0), (0, CH - 8)))
    b4 = jnp.pad(bias4, (0, CH - 8))
    rmat = jnp.repeat(jnp.eye(CEF, dtype=jnp.float32), CH, axis=1)

    h = jnp.pad(x, ((0, N_PAD - NN), (0, 0)))
    ci = _sc_cnt_inv(dst4, ones, zeros)
    layers = [
        (w1, root1, bias1, CIN, True),
        (w2, root2, bias2, CH, True),
        (w3_, root3, bias3, CH, True),
        (w4, r4, b4, CH, False),
    ]
    for wstk, root, bias, ch, relu in layers:
        hs = _sc_gather(h, src, ch)
        msg, hroot = _tc_msg(hs, ea, wstk, rmat, h,
                             root.astype(jnp.float32),
                             bias.reshape(1, CH).astype(jnp.float32), ch)
        h = _sc_scatter_update(msg, dst4, zeros, ci, hroot, relu)

    return h[:NN, :8]


# speedup vs baseline: 3.1567x; 1.0680x over previous
import functools

import jax
import jax.numpy as jnp
from jax import lax
from jax.experimental import pallas as pl
from jax.experimental.pallas import tpu as pltpu
from jax.experimental.pallas import tpu_sc as plsc

NN = 10000
EE = 80000
CIN = 32
CEF = 16
CH = 16

NW = 32
PW = 2560
E_PAD = NW * PW
PS16 = E_PAD // 16
SCH = PS16 // 128
N_PAD = 10240
PS = N_PAD // 16
UPS = N_PAD // 32


def _mesh_sc():
    return plsc.VectorSubcoreMesh(core_axis_name="c", subcore_axis_name="s",
                                  num_cores=2, num_subcores=16)


_SC_PARAMS = dict(
    compiler_params=pltpu.CompilerParams(use_tc_tiling_on_sc=False))


def _sc_gather(h, src, ch):
    @functools.partial(
        pl.kernel,
        out_type=jax.ShapeDtypeStruct((E_PAD, ch), jnp.float32),
        mesh=_mesh_sc(),
        scratch_types=[
            pltpu.VMEM((PW,), jnp.int32),
            pltpu.VMEM((PW, ch), jnp.float32),
            pltpu.SemaphoreType.DMA,
        ],
        **_SC_PARAMS,
    )
    def k(h_hbm, src_hbm, out_hbm, idx_v, rows_v, sem):
        wid = lax.axis_index("s") * 2 + lax.axis_index("c")
        base = wid * PW
        pltpu.sync_copy(src_hbm.at[pl.ds(base, PW)], idx_v)
        copies = []
        for j in range(PW // 128):
            copies.append(pltpu.async_copy(
                h_hbm.at[idx_v.at[pl.ds(j * 128, 128)]],
                rows_v.at[pl.ds(j * 128, 128)], sem))
        for cp in copies:
            cp.wait()
        pltpu.sync_copy(rows_v, out_hbm.at[pl.ds(base, PW)])

    return k(h, src)


def _sc_cnt_inv(dst4, ones, zeros):
    @functools.partial(
        pl.kernel,
        out_type=jax.ShapeDtypeStruct((N_PAD, CH), jnp.float32),
        mesh=_mesh_sc(),
        scratch_types=[
            pltpu.VMEM((SCH, 128), jnp.int32),
            pltpu.VMEM((128, CH), jnp.float32),
            pltpu.VMEM((UPS, CH), jnp.float32),
            pltpu.VMEM_SHARED((N_PAD, CH), jnp.float32),
        ],
        **_SC_PARAMS,
    )
    def k(dst_hbm, o_hbm, z_hbm, ci_hbm, idx_v, ones_v, buf_v, acc_sh):
        c = lax.axis_index("c")
        s = lax.axis_index("s")
        pltpu.sync_copy(z_hbm, acc_sh.at[pl.ds(s * PS, PS)])
        pltpu.sync_copy(dst_hbm.at[s], idx_v)
        pltpu.sync_copy(o_hbm, ones_v)
        plsc.subcore_barrier()

        def chunk5(step, _):
            for u in range(5):
                j = step * 5 + u
                pltpu.sync_copy(ones_v, acc_sh.at[idx_v.at[j]], add=True)
            return 0

        lax.fori_loop(0, SCH // 5, chunk5, 0)
        plsc.subcore_barrier()
        nbase = c * (N_PAD // 2) + s * UPS
        pltpu.sync_copy(acc_sh.at[pl.ds(nbase, UPS)], buf_v)

        def row(i, _):
            buf_v[i] = 1.0 / jnp.maximum(buf_v[i], 1.0)
            return 0

        lax.fori_loop(0, UPS, row, 0)
        pltpu.sync_copy(buf_v, ci_hbm.at[pl.ds(nbase, UPS)])

    return k(dst4, ones, zeros)


def _sc_fused(msg, dst4, zeros, ci, hroot, src, relu, do_gather):
    out_type = [jax.ShapeDtypeStruct((N_PAD, CH), jnp.float32)]
    if do_gather:
        out_type.append(jax.ShapeDtypeStruct((E_PAD, CH), jnp.float32))

    @functools.partial(
        pl.kernel,
        out_type=out_type,
        mesh=_mesh_sc(),
        scratch_types=[
            pltpu.VMEM((SCH, 128), jnp.int32),
            pltpu.VMEM((PS16, CH), jnp.float32),
            pltpu.VMEM((PS, CH), jnp.float32),
            pltpu.VMEM((PS, CH), jnp.float32),
            pltpu.VMEM((PS, CH), jnp.float32),
            pltpu.VMEM((PW,), jnp.int32),
            pltpu.VMEM_SHARED((N_PAD, CH), jnp.float32),
            pltpu.SemaphoreType.DMA,
        ],
        **_SC_PARAMS,
    )
    def k(msg_hbm, dst_hbm, z_hbm, ci_hbm, hr_hbm, src_hbm, *out_and_scratch):
        if do_gather:
            h_hbm, hs_hbm = out_and_scratch[:2]
            scratch = out_and_scratch[2:]
        else:
            h_hbm = out_and_scratch[0]
            scratch = out_and_scratch[1:]
        idx_v, rows_v, a_v, b_v, c_v, src_v, acc_sh, sem = scratch
        c = lax.axis_index("c")
        s = lax.axis_index("s")
        nb = s * PS
        pltpu.sync_copy(z_hbm, acc_sh.at[pl.ds(nb, PS)])
        pltpu.sync_copy(dst_hbm.at[s], idx_v)
        pltpu.sync_copy(msg_hbm.at[pl.ds(s * PS16, PS16)], rows_v)
        plsc.subcore_barrier()

        def chunk5(step, _):
            for u in range(5):
                j = step * 5 + u
                pltpu.async_copy(rows_v.at[pl.ds(j * 128, 128)],
                                 acc_sh.at[idx_v.at[j]], sem, add=True)
            return 0

        lax.fori_loop(0, SCH // 5, chunk5, 0)
        pltpu.make_async_copy(msg_hbm.at[pl.ds(0, PS16)], rows_v, sem).wait()
        plsc.subcore_barrier()
        pltpu.sync_copy(acc_sh.at[pl.ds(nb, PS)], a_v)
        pltpu.sync_copy(ci_hbm.at[pl.ds(nb, PS)], b_v)
        pltpu.sync_copy(hr_hbm.at[pl.ds(nb, PS)], c_v)

        def row4(step, _):
            for u in range(4):
                i = step * 4 + u
                v = a_v[i] * b_v[i] + c_v[i]
                a_v[i] = jnp.maximum(v, 0.0) if relu else v
            return 0

        lax.fori_loop(0, PS // 4, row4, 0)

        @pl.when((s // 8) == c)
        def _():
            pltpu.sync_copy(a_v, h_hbm.at[pl.ds(nb, PS)])

        if do_gather:
            pltpu.sync_copy(a_v, acc_sh.at[pl.ds(nb, PS)])
            plsc.subcore_barrier()
            wid = s * 2 + c
            base = wid * PW
            pltpu.sync_copy(src_hbm.at[pl.ds(base, PW)], src_v)
            for j in range(PW // 128):
                pltpu.async_copy(
                    acc_sh.at[src_v.at[pl.ds(j * 128, 128)]],
                    rows_v.at[pl.ds(j * 128, 128)], sem)
            pltpu.make_async_copy(msg_hbm.at[pl.ds(0, PW)],
                                  rows_v.at[pl.ds(0, PW)], sem).wait()
            pltpu.sync_copy(rows_v.at[pl.ds(0, PW)],
                            hs_hbm.at[pl.ds(base, PW)])

    return k(msg, dst4, zeros, ci, hroot, src)


def _tc_msg(hs, ea, wstk, rmat, h, root, bias, ch, tile_e=2048):
    grid = E_PAD // tile_e
    tile_n = N_PAD // grid

    def body(hs_ref, ea_ref, w_ref, r_ref, h_ref, rt_ref, b_ref,
             out_ref, hr_ref):
        t = jnp.dot(hs_ref[...], w_ref[...], preferred_element_type=jnp.float32)
        eae = jnp.dot(ea_ref[...], r_ref[...], preferred_element_type=jnp.float32)
        p = t[:, :256] * eae
        p = p[:, :128] + p[:, 128:256]
        p = p[:, :64] + p[:, 64:128]
        p = p[:, :32] + p[:, 32:64]
        out_ref[...] = p[:, :16] + p[:, 16:32] + t[:, 256:272]
        hr_ref[...] = jnp.dot(
            h_ref[...], rt_ref[...], preferred_element_type=jnp.float32) + b_ref[...]

    return pl.pallas_call(
        body,
        grid=(grid,),
        in_specs=[
            pl.BlockSpec((tile_e, ch), lambda i: (i, 0)),
            pl.BlockSpec((tile_e, CEF), lambda i: (i, 0)),
            pl.BlockSpec((ch, 272), lambda i: (0, 0)),
            pl.BlockSpec((CEF, 256), lambda i: (0, 0)),
            pl.BlockSpec((tile_n, ch), lambda i: (i, 0)),
            pl.BlockSpec((ch, CH), lambda i: (0, 0)),
            pl.BlockSpec((1, CH), lambda i: (0, 0)),
        ],
        out_specs=[
            pl.BlockSpec((tile_e, CH), lambda i: (i, 0)),
            pl.BlockSpec((tile_n, CH), lambda i: (i, 0)),
        ],
        out_shape=[
            jax.ShapeDtypeStruct((E_PAD, CH), jnp.float32),
            jax.ShapeDtypeStruct((N_PAD, CH), jnp.float32),
        ],
    )(hs, ea, wstk, rmat, h, root, bias)


def _prep_w(nw, nb, ch, out_ch):
    w3 = nw.T.reshape(CEF, ch, out_ch)
    nbm = nb.reshape(ch, out_ch)
    if out_ch < CH:
        w3 = jnp.pad(w3, ((0, 0), (0, 0), (0, CH - out_ch)))
        nbm = jnp.pad(nbm, ((0, 0), (0, CH - out_ch)))
    wstk = jnp.concatenate(
        [w3.transpose(1, 0, 2).reshape(ch, CEF * CH), nbm], axis=1)
    return wstk


def kernel(x, edge_index, edge_attr,
           nn1_w, nn1_b, root1, bias1,
           nn2_w, nn2_b, root2, bias2,
           nn3_w, nn3_b, root3, bias3,
           nn4_w, nn4_b, root4, bias4):
    src = jnp.pad(edge_index[0], (0, E_PAD - EE))
    dst4 = jnp.pad(edge_index[1], (0, E_PAD - EE),
                   constant_values=NN).reshape(16, SCH, 128)
    ea = jnp.pad(edge_attr, ((0, E_PAD - EE), (0, 0)))
    zeros = jnp.zeros((PS, CH), jnp.float32)
    ones = jnp.ones((128, CH), jnp.float32)

    w1 = _prep_w(nn1_w, nn1_b, CIN, CH)
    w2 = _prep_w(nn2_w, nn2_b, CH, CH)
    w3_ = _prep_w(nn3_w, nn3_b, CH, CH)
    w4 = _prep_w(nn4_w, nn4_b, CH, 8)
    r4 = jnp.pad(root4, ((0, 0), (0, CH - 8)))
    b4 = jnp.pad(bias4, (0, CH - 8))
    rmat = jnp.repeat(jnp.eye(CEF, dtype=jnp.float32), CH, axis=1)

    h = jnp.pad(x, ((0, N_PAD - NN), (0, 0)))
    ci = _sc_cnt_inv(dst4, ones, zeros)
    layers = [
        (w1, root1, bias1, CIN, True),
        (w2, root2, bias2, CH, True),
        (w3_, root3, bias3, CH, True),
        (w4, r4, b4, CH, False),
    ]
    hs = _sc_gather(h, src, CIN)
    for li, (wstk, root, bias, ch, relu) in enumerate(layers):
        msg, hroot = _tc_msg(hs, ea, wstk, rmat, h,
                             root.astype(jnp.float32),
                             bias.reshape(1, CH).astype(jnp.float32), ch)
        if li < 3:
            h, hs = _sc_fused(msg, dst4, zeros, ci, hroot, src, relu, True)
        else:
            (h,) = _sc_fused(msg, dst4, zeros, ci, hroot, src, relu, False)

    return h[:NN, :8]


# speedup vs baseline: 5.2608x; 1.6665x over previous
import functools

import jax
import jax.numpy as jnp
from jax import lax
from jax.experimental import pallas as pl
from jax.experimental.pallas import tpu as pltpu
from jax.experimental.pallas import tpu_sc as plsc

NN = 10000
EE = 80000
CIN = 32
CEF = 16
CH = 16

NW = 32
PW = 2560
E_PAD = NW * PW
PS16 = E_PAD // 16
SCH = PS16 // 128
N_PAD = 10240
PS = N_PAD // 16
UPS = N_PAD // 32


def _mesh_sc():
    return plsc.VectorSubcoreMesh(core_axis_name="c", subcore_axis_name="s",
                                  num_cores=2, num_subcores=16)


_SC_PARAMS = dict(
    compiler_params=pltpu.CompilerParams(use_tc_tiling_on_sc=False))


def _sc_gather2(xab, src):
    @functools.partial(
        pl.kernel,
        out_type=[jax.ShapeDtypeStruct((E_PAD, CH), jnp.float32),
                  jax.ShapeDtypeStruct((E_PAD, CH), jnp.float32)],
        mesh=_mesh_sc(),
        scratch_types=[
            pltpu.VMEM((PW,), jnp.int32),
            pltpu.VMEM((PW, CH), jnp.float32),
            pltpu.VMEM((PW, CH), jnp.float32),
            pltpu.SemaphoreType.DMA,
        ],
        **_SC_PARAMS,
    )
    def k(x_hbm, src_hbm, outa_hbm, outb_hbm, idx_v, rows_a, rows_b, sem):
        wid = lax.axis_index("s") * 2 + lax.axis_index("c")
        base = wid * PW
        pltpu.sync_copy(src_hbm.at[pl.ds(base, PW)], idx_v)
        for j in range(PW // 128):
            pltpu.async_copy(
                x_hbm.at[0].at[idx_v.at[pl.ds(j * 128, 128)]],
                rows_a.at[pl.ds(j * 128, 128)], sem)
            pltpu.async_copy(
                x_hbm.at[1].at[idx_v.at[pl.ds(j * 128, 128)]],
                rows_b.at[pl.ds(j * 128, 128)], sem)
        pltpu.make_async_copy(outa_hbm.at[pl.ds(0, PW)], rows_a, sem).wait()
        pltpu.make_async_copy(outb_hbm.at[pl.ds(0, PW)], rows_b, sem).wait()
        pltpu.sync_copy(rows_a, outa_hbm.at[pl.ds(base, PW)])
        pltpu.sync_copy(rows_b, outb_hbm.at[pl.ds(base, PW)])

    return k(xab, src)


def _sc_cnt_inv(dst4, ones, zeros):
    @functools.partial(
        pl.kernel,
        out_type=jax.ShapeDtypeStruct((N_PAD, CH), jnp.float32),
        mesh=_mesh_sc(),
        scratch_types=[
            pltpu.VMEM((SCH, 128), jnp.int32),
            pltpu.VMEM((128, CH), jnp.float32),
            pltpu.VMEM((UPS, CH), jnp.float32),
            pltpu.VMEM_SHARED((N_PAD, CH), jnp.float32),
        ],
        **_SC_PARAMS,
    )
    def k(dst_hbm, o_hbm, z_hbm, ci_hbm, idx_v, ones_v, buf_v, acc_sh):
        c = lax.axis_index("c")
        s = lax.axis_index("s")
        pltpu.sync_copy(z_hbm, acc_sh.at[pl.ds(s * PS, PS)])
        pltpu.sync_copy(dst_hbm.at[s], idx_v)
        pltpu.sync_copy(o_hbm, ones_v)
        plsc.subcore_barrier()

        def chunk5(step, _):
            for u in range(5):
                j = step * 5 + u
                pltpu.sync_copy(ones_v, acc_sh.at[idx_v.at[j]], add=True)
            return 0

        lax.fori_loop(0, SCH // 5, chunk5, 0)
        plsc.subcore_barrier()
        nbase = c * (N_PAD // 2) + s * UPS
        pltpu.sync_copy(acc_sh.at[pl.ds(nbase, UPS)], buf_v)

        def row(i, _):
            buf_v[i] = 1.0 / jnp.maximum(buf_v[i], 1.0)
            return 0

        lax.fori_loop(0, UPS, row, 0)
        pltpu.sync_copy(buf_v, ci_hbm.at[pl.ds(nbase, UPS)])

    return k(dst4, ones, zeros)


def _sc_fused(msg, dst4, zeros, ci, hroot, src, relu, do_gather):
    out_type = [jax.ShapeDtypeStruct((N_PAD, CH), jnp.float32)]
    if do_gather:
        out_type.append(jax.ShapeDtypeStruct((E_PAD, CH), jnp.float32))

    @functools.partial(
        pl.kernel,
        out_type=out_type,
        mesh=_mesh_sc(),
        scratch_types=[
            pltpu.VMEM((SCH, 128), jnp.int32),
            pltpu.VMEM((PS16, CH), jnp.float32),
            pltpu.VMEM((PS, CH), jnp.float32),
            pltpu.VMEM((PS, CH), jnp.float32),
            pltpu.VMEM((PS, CH), jnp.float32),
            pltpu.VMEM((PW,), jnp.int32),
            pltpu.VMEM_SHARED((N_PAD, CH), jnp.float32),
            pltpu.SemaphoreType.DMA,
        ],
        **_SC_PARAMS,
    )
    def k(msg_hbm, dst_hbm, z_hbm, ci_hbm, hr_hbm, src_hbm, *out_and_scratch):
        if do_gather:
            h_hbm, hs_hbm = out_and_scratch[:2]
            scratch = out_and_scratch[2:]
        else:
            h_hbm = out_and_scratch[0]
            scratch = out_and_scratch[1:]
        idx_v, rows_v, a_v, b_v, c_v, src_v, acc_sh, sem = scratch
        c = lax.axis_index("c")
        s = lax.axis_index("s")
        nb = s * PS
        pltpu.sync_copy(z_hbm, acc_sh.at[pl.ds(nb, PS)])
        pltpu.sync_copy(dst_hbm.at[s], idx_v)
        pltpu.sync_copy(msg_hbm.at[pl.ds(s * PS16, PS16)], rows_v)
        plsc.subcore_barrier()

        def chunk5(step, _):
            for u in range(5):
                j = step * 5 + u
                pltpu.async_copy(rows_v.at[pl.ds(j * 128, 128)],
                                 acc_sh.at[idx_v.at[j]], sem, add=True)
            return 0

        lax.fori_loop(0, SCH // 5, chunk5, 0)
        pltpu.make_async_copy(msg_hbm.at[pl.ds(0, PS16)], rows_v, sem).wait()
        plsc.subcore_barrier()
        pltpu.sync_copy(acc_sh.at[pl.ds(nb, PS)], a_v)
        pltpu.sync_copy(ci_hbm.at[pl.ds(nb, PS)], b_v)
        pltpu.sync_copy(hr_hbm.at[pl.ds(nb, PS)], c_v)

        def row4(step, _):
            for u in range(4):
                i = step * 4 + u
                v = a_v[i] * b_v[i] + c_v[i]
                a_v[i] = jnp.maximum(v, 0.0) if relu else v
            return 0

        lax.fori_loop(0, PS // 4, row4, 0)

        @pl.when((s // 8) == c)
        def _():
            pltpu.sync_copy(a_v, h_hbm.at[pl.ds(nb, PS)])

        if do_gather:
            pltpu.sync_copy(a_v, acc_sh.at[pl.ds(nb, PS)])
            plsc.subcore_barrier()
            wid = s * 2 + c
            base = wid * PW
            pltpu.sync_copy(src_hbm.at[pl.ds(base, PW)], src_v)
            for j in range(PW // 128):
                pltpu.async_copy(
                    acc_sh.at[src_v.at[pl.ds(j * 128, 128)]],
                    rows_v.at[pl.ds(j * 128, 128)], sem)
            pltpu.make_async_copy(msg_hbm.at[pl.ds(0, PW)],
                                  rows_v.at[pl.ds(0, PW)], sem).wait()
            pltpu.sync_copy(rows_v.at[pl.ds(0, PW)],
                            hs_hbm.at[pl.ds(base, PW)])

    return k(msg, dst4, zeros, ci, hroot, src)


EP8 = E_PAD // 8
NP8 = N_PAD // 8


def _tc_msg(hs_parts, ea8, w_parts, rblk, h_parts, root_parts, biasrow,
            tile_p=256):
    grid = EP8 // tile_p
    tile_n = NP8 // grid
    np_ = len(hs_parts)

    def body(*refs):
        hs_refs = refs[:np_]
        ea_ref = refs[np_]
        w_refs = refs[np_ + 1:2 * np_ + 1]
        r_ref = refs[2 * np_ + 1]
        h_refs = refs[2 * np_ + 2:3 * np_ + 2]
        rt_refs = refs[3 * np_ + 2:4 * np_ + 2]
        b_ref = refs[4 * np_ + 2]
        out_ref, hr_ref = refs[4 * np_ + 3:]
        t = jnp.dot(hs_refs[0][...], w_refs[0][...],
                    preferred_element_type=jnp.float32)
        hr = jnp.dot(h_refs[0][...], rt_refs[0][...],
                     preferred_element_type=jnp.float32)
        for q in range(1, np_):
            t = t + jnp.dot(hs_refs[q][...], w_refs[q][...],
                            preferred_element_type=jnp.float32)
            hr = hr + jnp.dot(h_refs[q][...], rt_refs[q][...],
                              preferred_element_type=jnp.float32)
        eae = jnp.dot(ea_ref[...], r_ref[...],
                      preferred_element_type=jnp.float32)
        p = t[:, :2048] * eae
        p = p[:, :1024] + p[:, 1024:2048]
        p = p[:, :512] + p[:, 512:1024]
        p = p[:, :256] + p[:, 256:512]
        out_ref[...] = p[:, :128] + p[:, 128:256] + t[:, 2048:2176]
        hr_ref[...] = hr + b_ref[...]

    in_specs = (
        [pl.BlockSpec((tile_p, 128), lambda i: (i, 0))] * np_
        + [pl.BlockSpec((tile_p, 128), lambda i: (i, 0))]
        + [pl.BlockSpec((128, 2176), lambda i: (0, 0))] * np_
        + [pl.BlockSpec((128, 2048), lambda i: (0, 0))]
        + [pl.BlockSpec((tile_n, 128), lambda i: (i, 0))] * np_
        + [pl.BlockSpec((128, 128), lambda i: (0, 0))] * np_
        + [pl.BlockSpec((1, 128), lambda i: (0, 0))]
    )
    return pl.pallas_call(
        body,
        grid=(grid,),
        in_specs=in_specs,
        out_specs=[
            pl.BlockSpec((tile_p, 128), lambda i: (i, 0)),
            pl.BlockSpec((tile_n, 128), lambda i: (i, 0)),
        ],
        out_shape=[
            jax.ShapeDtypeStruct((EP8, 128), jnp.float32),
            jax.ShapeDtypeStruct((NP8, 128), jnp.float32),
        ],
    )(*hs_parts, ea8, *w_parts, rblk, *h_parts, *root_parts, biasrow)


_EYE8 = None


def _prep_w(nw, nb, ch, out_ch):
    eye8 = jnp.eye(8, dtype=jnp.float32)
    w3 = nw.T.reshape(CEF, ch, out_ch)
    nbm = nb.reshape(ch, out_ch)
    if out_ch < CH:
        w3 = jnp.pad(w3, ((0, 0), (0, 0), (0, CH - out_ch)))
        nbm = jnp.pad(nbm, ((0, 0), (0, CH - out_ch)))
    parts = []
    for p in range(ch // CH):
        w3p = w3[:, p * CH:(p + 1) * CH, :]
        wblk = jnp.einsum('ij,fco->icfjo', eye8, w3p).reshape(128, CEF * 128)
        nbp = nbm[p * CH:(p + 1) * CH, :]
        nbblk = jnp.einsum('ij,co->icjo', eye8, nbp).reshape(128, 128)
        parts.append(jnp.concatenate([wblk, nbblk], axis=1))
    return parts


def _prep_root(root, bias, ch):
    eye8 = jnp.eye(8, dtype=jnp.float32)
    out_ch = root.shape[1]
    r = root.astype(jnp.float32)
    b = bias.astype(jnp.float32)
    if out_ch < CH:
        r = jnp.pad(r, ((0, 0), (0, CH - out_ch)))
        b = jnp.pad(b, (0, CH - out_ch))
    parts = []
    for p in range(ch // CH):
        rp = r[p * CH:(p + 1) * CH, :]
        parts.append(jnp.einsum('ij,co->icjo', eye8, rp).reshape(128, 128))
    biasrow = jnp.tile(b, 8)[None, :]
    return parts, biasrow


def _prep_rblk():
    eye8 = jnp.eye(8, dtype=jnp.float32)
    eye16 = jnp.eye(CEF, dtype=jnp.float32)
    r = jnp.einsum('ij,fg->ifgj', eye8, eye16)[..., None]
    r = r * jnp.ones((1, 1, 1, 1, CH), jnp.float32)
    return r.reshape(128, 2048)


def kernel(x, edge_index, edge_attr,
           nn1_w, nn1_b, root1, bias1,
           nn2_w, nn2_b, root2, bias2,
           nn3_w, nn3_b, root3, bias3,
           nn4_w, nn4_b, root4, bias4):
    src = jnp.pad(edge_index[0], (0, E_PAD - EE))
    dst4 = jnp.pad(edge_index[1], (0, E_PAD - EE),
                   constant_values=NN).reshape(16, SCH, 128)
    ea8 = jnp.pad(edge_attr.reshape(EE * CEF // 128, 128),
                  ((0, EP8 - EE * CEF // 128), (0, 0)))
    zeros = jnp.zeros((PS, CH), jnp.float32)
    ones = jnp.ones((128, CH), jnp.float32)

    w1 = _prep_w(nn1_w, nn1_b, CIN, CH)
    w2 = _prep_w(nn2_w, nn2_b, CH, CH)
    w3_ = _prep_w(nn3_w, nn3_b, CH, CH)
    w4 = _prep_w(nn4_w, nn4_b, CH, 8)
    rt1, br1 = _prep_root(root1, bias1, CIN)
    rt2, br2 = _prep_root(root2, bias2, CH)
    rt3, br3 = _prep_root(root3, bias3, CH)
    rt4, br4 = _prep_root(root4, bias4, CH)
    rblk = _prep_rblk()

    xp = jnp.pad(x, ((0, N_PAD - NN), (0, 0)))
    xab = jnp.stack([xp[:, :CH], xp[:, CH:]])
    ci = _sc_cnt_inv(dst4, ones, zeros)
    hsa, hsb = _sc_gather2(xab, src)
    hs_parts = [hsa.reshape(EP8, 128), hsb.reshape(EP8, 128)]
    h_parts = [xab[0].reshape(NP8, 128), xab[1].reshape(NP8, 128)]
    layers = [
        (w1, rt1, br1, True),
        (w2, rt2, br2, True),
        (w3_, rt3, br3, True),
        (w4, rt4, br4, False),
    ]
    h = None
    for li, (wp, rtp, brow, relu) in enumerate(layers):
        msg8, hroot8 = _tc_msg(hs_parts, ea8, wp, rblk, h_parts, rtp, brow)
        msg = msg8.reshape(E_PAD, CH)
        hroot = hroot8.reshape(N_PAD, CH)
        if li < 3:
            h, hs = _sc_fused(msg, dst4, zeros, ci, hroot, src, relu, True)
            hs_parts = [hs.reshape(EP8, 128)]
            h_parts = [h.reshape(NP8, 128)]
        else:
            (h,) = _sc_fused(msg, dst4, zeros, ci, hroot, src, relu, False)

    return h[:NN, :8]


# speedup vs baseline: 5.4127x; 1.0289x over previous
import functools

import jax
import jax.numpy as jnp
from jax import lax
from jax.experimental import pallas as pl
from jax.experimental.pallas import tpu as pltpu
from jax.experimental.pallas import tpu_sc as plsc

NN = 10000
EE = 80000
CIN = 32
CEF = 16
CH = 16

NW = 32
PW = 2560
E_PAD = NW * PW
PS16 = E_PAD // 16
SCH = PS16 // 128
N_PAD = 10240
PS = N_PAD // 16
UPS = N_PAD // 32


def _mesh_sc():
    return plsc.VectorSubcoreMesh(core_axis_name="c", subcore_axis_name="s",
                                  num_cores=2, num_subcores=16)


_SC_PARAMS = dict(
    compiler_params=pltpu.CompilerParams(use_tc_tiling_on_sc=False))


def _sc_gather2(xab, src):
    @functools.partial(
        pl.kernel,
        out_type=[jax.ShapeDtypeStruct((E_PAD, CH), jnp.float32),
                  jax.ShapeDtypeStruct((E_PAD, CH), jnp.float32)],
        mesh=_mesh_sc(),
        scratch_types=[
            pltpu.VMEM((PW,), jnp.int32),
            pltpu.VMEM((PW, CH), jnp.float32),
            pltpu.VMEM((PW, CH), jnp.float32),
            pltpu.SemaphoreType.DMA,
        ],
        **_SC_PARAMS,
    )
    def k(x_hbm, src_hbm, outa_hbm, outb_hbm, idx_v, rows_a, rows_b, sem):
        wid = lax.axis_index("s") * 2 + lax.axis_index("c")
        base = wid * PW
        pltpu.sync_copy(src_hbm.at[pl.ds(base, PW)], idx_v)
        for j in range(PW // 128):
            pltpu.async_copy(
                x_hbm.at[0].at[idx_v.at[pl.ds(j * 128, 128)]],
                rows_a.at[pl.ds(j * 128, 128)], sem)
            pltpu.async_copy(
                x_hbm.at[1].at[idx_v.at[pl.ds(j * 128, 128)]],
                rows_b.at[pl.ds(j * 128, 128)], sem)
        pltpu.make_async_copy(outa_hbm.at[pl.ds(0, PW)], rows_a, sem).wait()
        pltpu.make_async_copy(outb_hbm.at[pl.ds(0, PW)], rows_b, sem).wait()
        pltpu.sync_copy(rows_a, outa_hbm.at[pl.ds(base, PW)])
        pltpu.sync_copy(rows_b, outb_hbm.at[pl.ds(base, PW)])

    return k(xab, src)


def _sc_cnt_inv(dst4, ones, zeros):
    @functools.partial(
        pl.kernel,
        out_type=jax.ShapeDtypeStruct((N_PAD, CH), jnp.float32),
        mesh=_mesh_sc(),
        scratch_types=[
            pltpu.VMEM((SCH, 128), jnp.int32),
            pltpu.VMEM((128, CH), jnp.float32),
            pltpu.VMEM((UPS, CH), jnp.float32),
            pltpu.VMEM_SHARED((N_PAD, CH), jnp.float32),
        ],
        **_SC_PARAMS,
    )
    def k(dst_hbm, o_hbm, z_hbm, ci_hbm, idx_v, ones_v, buf_v, acc_sh):
        c = lax.axis_index("c")
        s = lax.axis_index("s")
        pltpu.sync_copy(z_hbm, acc_sh.at[pl.ds(s * PS, PS)])
        pltpu.sync_copy(dst_hbm.at[s], idx_v)
        pltpu.sync_copy(o_hbm, ones_v)
        plsc.subcore_barrier()

        def chunk5(step, _):
            for u in range(5):
                j = step * 5 + u
                pltpu.sync_copy(ones_v, acc_sh.at[idx_v.at[j]], add=True)
            return 0

        lax.fori_loop(0, SCH // 5, chunk5, 0)
        plsc.subcore_barrier()
        nbase = c * (N_PAD // 2) + s * UPS
        pltpu.sync_copy(acc_sh.at[pl.ds(nbase, UPS)], buf_v)

        def row(i, _):
            buf_v[i] = 1.0 / jnp.maximum(buf_v[i], 1.0)
            return 0

        lax.fori_loop(0, UPS, row, 0)
        pltpu.sync_copy(buf_v, ci_hbm.at[pl.ds(nbase, UPS)])

    return k(dst4, ones, zeros)


def _sc_fused(msg, dst4, zeros, ci, hroot, src, relu, do_gather):
    out_type = [jax.ShapeDtypeStruct((N_PAD, CH), jnp.float32)]
    if do_gather:
        out_type.append(jax.ShapeDtypeStruct((E_PAD, CH), jnp.float32))

    @functools.partial(
        pl.kernel,
        out_type=out_type,
        mesh=_mesh_sc(),
        scratch_types=[
            pltpu.VMEM((SCH, 128), jnp.int32),
            pltpu.VMEM((PS16, CH), jnp.float32),
            pltpu.VMEM((PS, CH), jnp.float32),
            pltpu.VMEM((PS, CH), jnp.float32),
            pltpu.VMEM((PS, CH), jnp.float32),
            pltpu.VMEM((PW,), jnp.int32),
            pltpu.VMEM_SHARED((N_PAD, CH), jnp.float32),
            pltpu.SemaphoreType.DMA,
            pltpu.SemaphoreType.DMA,
        ],
        **_SC_PARAMS,
    )
    def k(msg_hbm, dst_hbm, z_hbm, ci_hbm, hr_hbm, src_hbm, *out_and_scratch):
        if do_gather:
            h_hbm, hs_hbm = out_and_scratch[:2]
            scratch = out_and_scratch[2:]
        else:
            h_hbm = out_and_scratch[0]
            scratch = out_and_scratch[1:]
        idx_v, rows_v, a_v, b_v, c_v, src_v, acc_sh, sem, sem2 = scratch
        c = lax.axis_index("c")
        s = lax.axis_index("s")
        nb = s * PS
        d_z = pltpu.async_copy(z_hbm, acc_sh.at[pl.ds(nb, PS)], sem2)
        d_i = pltpu.async_copy(dst_hbm.at[s], idx_v, sem2)
        d_m = pltpu.async_copy(msg_hbm.at[pl.ds(s * PS16, PS16)], rows_v, sem2)
        d_ci = pltpu.async_copy(ci_hbm.at[pl.ds(nb, PS)], b_v, sem2)
        d_hr = pltpu.async_copy(hr_hbm.at[pl.ds(nb, PS)], c_v, sem2)
        d_z.wait()
        d_i.wait()
        d_m.wait()
        plsc.subcore_barrier()

        def chunk5(step, _):
            for u in range(5):
                j = step * 5 + u
                pltpu.async_copy(rows_v.at[pl.ds(j * 128, 128)],
                                 acc_sh.at[idx_v.at[j]], sem, add=True)
            return 0

        lax.fori_loop(0, SCH // 5, chunk5, 0)
        pltpu.make_async_copy(msg_hbm.at[pl.ds(0, PS16)], rows_v, sem).wait()
        plsc.subcore_barrier()
        d_ci.wait()
        d_hr.wait()
        pltpu.sync_copy(acc_sh.at[pl.ds(nb, PS)], a_v)

        def row4(step, _):
            for u in range(4):
                i = step * 4 + u
                v = a_v[i] * b_v[i] + c_v[i]
                a_v[i] = jnp.maximum(v, 0.0) if relu else v
            return 0

        lax.fori_loop(0, PS // 4, row4, 0)

        @pl.when((s // 8) == c)
        def _():
            pltpu.sync_copy(a_v, h_hbm.at[pl.ds(nb, PS)])

        if do_gather:
            pltpu.sync_copy(a_v, acc_sh.at[pl.ds(nb, PS)])
            plsc.subcore_barrier()
            wid = s * 2 + c
            base = wid * PW
            pltpu.sync_copy(src_hbm.at[pl.ds(base, PW)], src_v)
            for j in range(PW // 128):
                pltpu.async_copy(
                    acc_sh.at[src_v.at[pl.ds(j * 128, 128)]],
                    rows_v.at[pl.ds(j * 128, 128)], sem)
            pltpu.make_async_copy(msg_hbm.at[pl.ds(0, PW)],
                                  rows_v.at[pl.ds(0, PW)], sem).wait()
            pltpu.sync_copy(rows_v.at[pl.ds(0, PW)],
                            hs_hbm.at[pl.ds(base, PW)])

    return k(msg, dst4, zeros, ci, hroot, src)


EP8 = E_PAD // 8
NP8 = N_PAD // 8


def _tc_msg(hs_parts, ea8, w_parts, rblk, h_parts, root_parts, biasrow,
            tile_p=256):
    grid = EP8 // tile_p
    tile_n = NP8 // grid
    np_ = len(hs_parts)

    def body(*refs):
        hs_refs = refs[:np_]
        ea_ref = refs[np_]
        w_refs = refs[np_ + 1:2 * np_ + 1]
        r_ref = refs[2 * np_ + 1]
        h_refs = refs[2 * np_ + 2:3 * np_ + 2]
        rt_refs = refs[3 * np_ + 2:4 * np_ + 2]
        b_ref = refs[4 * np_ + 2]
        out_ref, hr_ref = refs[4 * np_ + 3:]
        t = jnp.dot(hs_refs[0][...], w_refs[0][...],
                    preferred_element_type=jnp.float32)
        hr = jnp.dot(h_refs[0][...], rt_refs[0][...],
                     preferred_element_type=jnp.float32)
        for q in range(1, np_):
            t = t + jnp.dot(hs_refs[q][...], w_refs[q][...],
                            preferred_element_type=jnp.float32)
            hr = hr + jnp.dot(h_refs[q][...], rt_refs[q][...],
                              preferred_element_type=jnp.float32)
        eae = jnp.dot(ea_ref[...], r_ref[...],
                      preferred_element_type=jnp.float32)
        p = t[:, :2048] * eae
        p = p[:, :1024] + p[:, 1024:2048]
        p = p[:, :512] + p[:, 512:1024]
        p = p[:, :256] + p[:, 256:512]
        out_ref[...] = p[:, :128] + p[:, 128:256] + t[:, 2048:2176]
        hr_ref[...] = hr + b_ref[...]

    in_specs = (
        [pl.BlockSpec((tile_p, 128), lambda i: (i, 0))] * np_
        + [pl.BlockSpec((tile_p, 128), lambda i: (i, 0))]
        + [pl.BlockSpec((128, 2176), lambda i: (0, 0))] * np_
        + [pl.BlockSpec((128, 2048), lambda i: (0, 0))]
        + [pl.BlockSpec((tile_n, 128), lambda i: (i, 0))] * np_
        + [pl.BlockSpec((128, 128), lambda i: (0, 0))] * np_
        + [pl.BlockSpec((1, 128), lambda i: (0, 0))]
    )
    return pl.pallas_call(
        body,
        grid=(grid,),
        in_specs=in_specs,
        out_specs=[
            pl.BlockSpec((tile_p, 128), lambda i: (i, 0)),
            pl.BlockSpec((tile_n, 128), lambda i: (i, 0)),
        ],
        out_shape=[
            jax.ShapeDtypeStruct((EP8, 128), jnp.float32),
            jax.ShapeDtypeStruct((NP8, 128), jnp.float32),
        ],
    )(*hs_parts, ea8, *w_parts, rblk, *h_parts, *root_parts, biasrow)


_EYE8 = None


def _prep_w(nw, nb, ch, out_ch):
    eye8 = jnp.eye(8, dtype=jnp.float32)
    w3 = nw.T.reshape(CEF, ch, out_ch)
    nbm = nb.reshape(ch, out_ch)
    if out_ch < CH:
        w3 = jnp.pad(w3, ((0, 0), (0, 0), (0, CH - out_ch)))
        nbm = jnp.pad(nbm, ((0, 0), (0, CH - out_ch)))
    parts = []
    for p in range(ch // CH):
        w3p = w3[:, p * CH:(p + 1) * CH, :]
        wblk = jnp.einsum('ij,fco->icfjo', eye8, w3p).reshape(128, CEF * 128)
        nbp = nbm[p * CH:(p + 1) * CH, :]
        nbblk = jnp.einsum('ij,co->icjo', eye8, nbp).reshape(128, 128)
        parts.append(jnp.concatenate([wblk, nbblk], axis=1))
    return parts


def _prep_root(root, bias, ch):
    eye8 = jnp.eye(8, dtype=jnp.float32)
    out_ch = root.shape[1]
    r = root.astype(jnp.float32)
    b = bias.astype(jnp.float32)
    if out_ch < CH:
        r = jnp.pad(r, ((0, 0), (0, CH - out_ch)))
        b = jnp.pad(b, (0, CH - out_ch))
    parts = []
    for p in range(ch // CH):
        rp = r[p * CH:(p + 1) * CH, :]
        parts.append(jnp.einsum('ij,co->icjo', eye8, rp).reshape(128, 128))
    biasrow = jnp.tile(b, 8)[None, :]
    return parts, biasrow


def _prep_rblk():
    eye8 = jnp.eye(8, dtype=jnp.float32)
    eye16 = jnp.eye(CEF, dtype=jnp.float32)
    r = jnp.einsum('ij,fg->ifgj', eye8, eye16)[..., None]
    r = r * jnp.ones((1, 1, 1, 1, CH), jnp.float32)
    return r.reshape(128, 2048)


def kernel(x, edge_index, edge_attr,
           nn1_w, nn1_b, root1, bias1,
           nn2_w, nn2_b, root2, bias2,
           nn3_w, nn3_b, root3, bias3,
           nn4_w, nn4_b, root4, bias4):
    src = jnp.pad(edge_index[0], (0, E_PAD - EE))
    dst4 = jnp.pad(edge_index[1], (0, E_PAD - EE),
                   constant_values=NN).reshape(16, SCH, 128)
    ea8 = jnp.pad(edge_attr.reshape(EE * CEF // 128, 128),
                  ((0, EP8 - EE * CEF // 128), (0, 0)))
    zeros = jnp.zeros((PS, CH), jnp.float32)
    ones = jnp.ones((128, CH), jnp.float32)

    w1 = _prep_w(nn1_w, nn1_b, CIN, CH)
    w2 = _prep_w(nn2_w, nn2_b, CH, CH)
    w3_ = _prep_w(nn3_w, nn3_b, CH, CH)
    w4 = _prep_w(nn4_w, nn4_b, CH, 8)
    rt1, br1 = _prep_root(root1, bias1, CIN)
    rt2, br2 = _prep_root(root2, bias2, CH)
    rt3, br3 = _prep_root(root3, bias3, CH)
    rt4, br4 = _prep_root(root4, bias4, CH)
    rblk = _prep_rblk()

    xp = jnp.pad(x, ((0, N_PAD - NN), (0, 0)))
    xab = jnp.stack([xp[:, :CH], xp[:, CH:]])
    ci = _sc_cnt_inv(dst4, ones, zeros)
    hsa, hsb = _sc_gather2(xab, src)
    hs_parts = [hsa.reshape(EP8, 128), hsb.reshape(EP8, 128)]
    h_parts = [xab[0].reshape(NP8, 128), xab[1].reshape(NP8, 128)]
    layers = [
        (w1, rt1, br1, True),
        (w2, rt2, br2, True),
        (w3_, rt3, br3, True),
        (w4, rt4, br4, False),
    ]
    h = None
    for li, (wp, rtp, brow, relu) in enumerate(layers):
        msg8, hroot8 = _tc_msg(hs_parts, ea8, wp, rblk, h_parts, rtp, brow)
        msg = msg8.reshape(E_PAD, CH)
        hroot = hroot8.reshape(N_PAD, CH)
        if li < 3:
            h, hs = _sc_fused(msg, dst4, zeros, ci, hroot, src, relu, True)
            hs_parts = [hs.reshape(EP8, 128)]
            h_parts = [h.reshape(NP8, 128)]
        else:
            (h,) = _sc_fused(msg, dst4, zeros, ci, hroot, src, relu, False)

    return h[:NN, :8]
